# Initial kernel scaffold; baseline (speedup 1.0000x reference)
import jax
import jax.numpy as jnp
from jax.experimental import pallas as pl


def kernel(X, Q, ranks):
    raise NotImplementedError("write your pallas kernel here")


# speedup vs baseline: 19.2492x; 19.2492x over previous
import functools

import jax
import jax.numpy as jnp
from jax import lax
from jax.experimental import pallas as pl
from jax.experimental.pallas import tpu as pltpu
from jax.experimental.pallas import tpu_sc as plsc

_M = 400
_K = 10
_BETA = 0.15
_D = 64
_NQ = 256
_CH = 128
_DP = 128


def _sc_gather(table, idx3d, n_workers, n_chunks):
    mesh = plsc.VectorSubcoreMesh(core_axis_name="c", subcore_axis_name="s")
    num_cores = 2

    @functools.partial(
        pl.kernel,
        mesh=mesh,
        out_type=jax.ShapeDtypeStruct((n_workers, n_chunks, _CH, _DP), jnp.float32),
        scratch_types=[
            pltpu.VMEM((n_chunks, _CH), jnp.int32),
            pltpu.VMEM((_CH, _DP), jnp.float32),
            pltpu.VMEM((_CH, _DP), jnp.float32),
            pltpu.SemaphoreType.DMA,
            pltpu.SemaphoreType.DMA,
        ],
    )
    def gather_kernel(table_hbm, idx_hbm, out_hbm, idx_v, rows0, rows1, sem0, sem1):
        wid = lax.axis_index("s") * num_cores + lax.axis_index("c")
        pltpu.sync_copy(idx_hbm.at[wid], idx_v)
        pltpu.async_copy(table_hbm.at[idx_v.at[0]], rows0, sem0)

        def body(c, _):
            even = lax.rem(c, 2) == 0

            @pl.when(even)
            def _():
                pltpu.make_async_copy(table_hbm.at[idx_v.at[c]], rows0, sem0).wait()

                @pl.when(c + 1 < n_chunks)
                def _():
                    pltpu.async_copy(table_hbm.at[idx_v.at[c + 1]], rows1, sem1)

                pltpu.sync_copy(rows0, out_hbm.at[wid, c])

            @pl.when(jnp.logical_not(even))
            def _():
                pltpu.make_async_copy(table_hbm.at[idx_v.at[c]], rows1, sem1).wait()

                @pl.when(c + 1 < n_chunks)
                def _():
                    pltpu.async_copy(table_hbm.at[idx_v.at[c + 1]], rows0, sem0)

                pltpu.sync_copy(rows1, out_hbm.at[wid, c])

            return _

        lax.fori_loop(0, n_chunks, body, None)

    return gather_kernel(table, idx3d)


def _rerank_body(x1_ref, q_ref, rt_ref, rerank_ref, score_ref, pre_ref, xdba_ref):
    x1 = x1_ref[0][:, 0:_D]
    g = lax.dot_general(x1, x1, (((1,), (1,)), ((), ())),
                        preferred_element_type=jnp.float32)
    iota_l = lax.broadcasted_iota(jnp.int32, (_M, _M), 1)
    iota_s = lax.broadcasted_iota(jnp.int32, (_M, _M), 0)

    h0 = x1.astype(jnp.bfloat16).astype(jnp.float32)
    r1 = x1 - h0
    h1 = r1.astype(jnp.bfloat16).astype(jnp.float32)
    h2 = r1 - h1

    acc = jnp.zeros((_M, _D), jnp.float32)
    den = jnp.zeros((_M, 1), jnp.float32)
    for k in range(_K):
        mval = jnp.max(g, axis=1, keepdims=True)
        idx = jnp.min(jnp.where(g == mval, iota_l, _M), axis=1, keepdims=True)
        onehot = iota_l == idx
        oh = onehot.astype(jnp.float32)
        sel = (lax.dot_general(oh, h0, (((1,), (0,)), ((), ())),
                               preferred_element_type=jnp.float32)
               + lax.dot_general(oh, h1, (((1,), (0,)), ((), ())),
                                 preferred_element_type=jnp.float32)
               + lax.dot_general(oh, h2, (((1,), (0,)), ((), ())),
                                 preferred_element_type=jnp.float32))
        if k == 0:
            w = jnp.ones((_M, 1), jnp.float32)
            acc = sel
            den = w
        else:
            w = _BETA * mval
            acc = acc + sel * w
            den = den + w
        g = jnp.where(onehot, -jnp.inf, g)

    xd = acc * (1.0 / den)
    xdba_ref[0] = xd

    q2 = q_ref[0]
    qrep = jnp.broadcast_to(q2, (8, _D))
    s8 = lax.dot_general(xd, qrep, (((1,), (1,)), ((), ())),
                         preferred_element_type=jnp.float32)
    s_col = s8[:, 0:1]
    s_row = jnp.transpose(s_col, (1, 0))
    score_ref[0] = s_row

    s_bj = jnp.broadcast_to(s_row, (_M, _M))
    s_bi = jnp.broadcast_to(s_col, (_M, _M))
    gt = (s_bj > s_bi).astype(jnp.int32)
    eq = jnp.logical_and(s_bj == s_bi, iota_l < iota_s).astype(jnp.int32)
    rank = jnp.sum(gt + eq, axis=1, keepdims=True)

    p_mat = rank == iota_l
    pre_ref[0] = jnp.sum(jnp.where(p_mat, iota_s, 0), axis=0, keepdims=True)
    rt_col = jnp.transpose(rt_ref[0], (1, 0))
    rt_b = jnp.broadcast_to(rt_col, (_M, _M))
    rerank_ref[0] = jnp.sum(jnp.where(p_mat, rt_b, 0), axis=0, keepdims=True)


def _tc_rerank(x1, q3, rt3):
    out_shapes = (
        jax.ShapeDtypeStruct((_NQ, 1, _M), jnp.int32),
        jax.ShapeDtypeStruct((_NQ, 1, _M), jnp.float32),
        jax.ShapeDtypeStruct((_NQ, 1, _M), jnp.int32),
        jax.ShapeDtypeStruct((_NQ, _M, _D), jnp.float32),
    )
    return pl.pallas_call(
        _rerank_body,
        grid=(_NQ,),
        in_specs=[
            pl.BlockSpec((1, _M, _DP), lambda i: (i, 0, 0)),
            pl.BlockSpec((1, 1, _D), lambda i: (i, 0, 0)),
            pl.BlockSpec((1, 1, _M), lambda i: (i, 0, 0)),
        ],
        out_specs=(
            pl.BlockSpec((1, 1, _M), lambda i: (i, 0, 0)),
            pl.BlockSpec((1, 1, _M), lambda i: (i, 0, 0)),
            pl.BlockSpec((1, 1, _M), lambda i: (i, 0, 0)),
            pl.BlockSpec((1, _M, _D), lambda i: (i, 0, 0)),
        ),
        out_shape=out_shapes,
        compiler_params=pltpu.CompilerParams(
            dimension_semantics=("parallel",),
        ),
    )(x1, q3, rt3)


def kernel(X, Q, ranks):
    rt = jnp.transpose(ranks[:_M, :])
    n_workers = 32
    total = _NQ * _M
    n_chunks = total // (n_workers * _CH)
    idx3d = rt.reshape(n_workers, n_chunks, _CH)
    x_pad = jnp.pad(X, ((0, 0), (0, _DP - _D)))
    x1 = _sc_gather(x_pad, idx3d, n_workers, n_chunks).reshape(_NQ, _M, _DP)
    rerank3, score3, pre3, xdba = _tc_rerank(
        x1, Q.reshape(_NQ, 1, _D), rt.reshape(_NQ, 1, _M))
    return (rerank3.reshape(_NQ, _M), score3.reshape(_NQ, _M),
            pre3.reshape(_NQ, _M), xdba)
